# baseline (speedup 1.0000x reference)
import functools

import jax
import jax.numpy as jnp
from jax import lax
from jax.experimental import pallas as pl
from jax.experimental.pallas import tpu as pltpu
from jax.experimental.pallas import tpu_sc as plsc

HIDDEN = 64
NSTATES = 8
SEQ = 197
EPS = 1e-5

NC, NS = 2, 16
NW = NC * NS
RCH = 480
GSL = [(o, min(128, RCH - o)) for o in range(0, RCH, 128)]


def _prep_body(g_ref, st_ref, pos_ref, gam_ref, bet_ref, plut_ref, idx_ref):
    pid = pl.program_id(0)
    g = g_ref[...]
    nb = g.shape[0]
    ids = jnp.concatenate([jnp.zeros((nb, 1), jnp.int32), g], axis=1)
    col = lax.broadcasted_iota(jnp.int32, (nb, SEQ), 1)
    idx_ref[...] = col * NSTATES + ids

    @pl.when(pid == 0)
    def _():
        x = (pos_ref[0:SEQ, :].reshape(SEQ, 1, HIDDEN)
             + st_ref[...].reshape(1, NSTATES, HIDDEN))
        mu = jnp.mean(x, axis=-1, keepdims=True)
        var = jnp.mean((x - mu) ** 2, axis=-1, keepdims=True)
        y = (x - mu) / jnp.sqrt(var + EPS)
        lut = (y * gam_ref[...].reshape(1, 1, HIDDEN)
               + bet_ref[...].reshape(1, 1, HIDDEN))
        lut_next = jnp.concatenate([lut[1:SEQ], lut[0:1]], axis=0)
        plut_ref[:, :, :, 0:HIDDEN] = jnp.broadcast_to(
            lut.reshape(SEQ, NSTATES, 1, HIDDEN),
            (SEQ, NSTATES, NSTATES, HIDDEN))
        plut_ref[:, :, :, HIDDEN:] = jnp.broadcast_to(
            lut_next.reshape(SEQ, 1, NSTATES, HIDDEN),
            (SEQ, NSTATES, NSTATES, HIDDEN))


def _prep(grid_cat, states_table, pos_table, gamma2, beta2):
    nrows = grid_cat.shape[0]
    nblk = 4
    blk = nrows // nblk
    return pl.pallas_call(
        _prep_body,
        grid=(nblk,),
        in_specs=[
            pl.BlockSpec((blk, grid_cat.shape[1]), lambda i: (i, 0)),
            pl.BlockSpec(states_table.shape, lambda i: (0, 0)),
            pl.BlockSpec(pos_table.shape, lambda i: (0, 0)),
            pl.BlockSpec((1, HIDDEN), lambda i: (0, 0)),
            pl.BlockSpec((1, HIDDEN), lambda i: (0, 0)),
        ],
        out_specs=[
            pl.BlockSpec((SEQ, NSTATES, NSTATES, 2 * HIDDEN),
                         lambda i: (0, 0, 0, 0)),
            pl.BlockSpec((blk, SEQ), lambda i: (i, 0)),
        ],
        out_shape=[
            jax.ShapeDtypeStruct((SEQ, NSTATES, NSTATES, 2 * HIDDEN),
                                 jnp.float32),
            jax.ShapeDtypeStruct((nrows, SEQ), jnp.int32),
        ],
    )(grid_cat, states_table, pos_table, gamma2, beta2)


def _sc_gather(plut, pidx, batch):
    pairs_per_tile = (batch * SEQ // 2) // NW
    nfull = (pairs_per_tile // RCH) & ~1
    tail = pairs_per_tile - nfull * RCH
    tail_gsl = [(o, min(128, tail - o)) for o in range(0, tail, 128)]
    mesh = plsc.VectorSubcoreMesh(core_axis_name="c", subcore_axis_name="s")
    out_t = jax.ShapeDtypeStruct((batch * SEQ // 2, 2 * HIDDEN), jnp.float32)

    @functools.partial(
        pl.kernel,
        out_type=(out_t, out_t),
        mesh=mesh,
        compiler_params=pltpu.CompilerParams(use_tc_tiling_on_sc=False),
        scratch_types=[
            pltpu.VMEM((RCH,), jnp.int32),
            pltpu.VMEM((RCH,), jnp.int32),
            pltpu.VMEM((RCH, 2 * HIDDEN), jnp.float32),
            pltpu.VMEM((RCH, 2 * HIDDEN), jnp.float32),
            pltpu.SemaphoreType.DMA,
            pltpu.SemaphoreType.DMA,
            pltpu.SemaphoreType.DMA,
            pltpu.SemaphoreType.DMA,
            pltpu.SemaphoreType.DMA,
            pltpu.SemaphoreType.DMA,
        ],
    )
    def run(plut_hbm, pidx_hbm, out_si, out_sj,
            iv0, iv1, rv0, rv1, si0, si1, sg0, sg1, ss0, ss1):
        wid = lax.axis_index("s") * NC + lax.axis_index("c")
        idx_v = (iv0, iv1)
        rows_v = (rv0, rv1)
        semi = (si0, si1)
        semg = (sg0, sg1)
        sems = (ss0, ss1)

        def run_half(out_ref, half):
            hrow0 = wid * pairs_per_tile
            irow0 = half * batch * SEQ // 2 + wid * pairs_per_tile

            def fire_idx(ci, b, n):
                pltpu.async_copy(pidx_hbm.at[pl.ds(irow0 + ci * RCH, n)],
                                 idx_v[b].at[pl.ds(0, n)], semi[b])

            def wait_idx(b, n):
                pltpu.make_async_copy(pidx_hbm.at[pl.ds(0, n)],
                                      idx_v[b].at[pl.ds(0, n)], semi[b]).wait()

            def fire_gathers(b, gsl):
                for (o, sz) in gsl:
                    pltpu.async_copy(plut_hbm.at[idx_v[b].at[pl.ds(o, sz)]],
                                     rows_v[b].at[pl.ds(o, sz)], semg[b])

            def wait_gathers(b, n):
                pltpu.make_async_copy(
                    plut_hbm.at[pl.ds(0, n)],
                    rows_v[b].at[pl.ds(0, n)], semg[b]).wait()

            def fire_scatter(ci, b, n):
                pltpu.async_copy(rows_v[b].at[pl.ds(0, n)],
                                 out_ref.at[pl.ds(hrow0 + ci * RCH, n)],
                                 sems[b])

            def drain_scatter(b, n):
                pltpu.make_async_copy(rows_v[b].at[pl.ds(0, n)],
                                      out_ref.at[pl.ds(0, n)], sems[b]).wait()

            fire_idx(0, 0, RCH)
            fire_idx(1, 1, RCH)

            @pl.loop(0, nfull, step=2)
            def _(g):
                for b in range(2):
                    gi = g + b
                    wait_idx(b, RCH)

                    @pl.when(gi >= 2)
                    def _():
                        drain_scatter(b, RCH)

                    fire_gathers(b, GSL)
                    wait_gathers(b, RCH)

                    @pl.when(gi + 2 < nfull)
                    def _():
                        fire_idx(gi + 2, b, RCH)

                    fire_scatter(gi, b, RCH)

            if tail:
                fire_idx(nfull, 0, tail)
                wait_idx(0, tail)
                drain_scatter(0, RCH)
                fire_gathers(0, tail_gsl)
                wait_gathers(0, tail)
                drain_scatter(1, RCH)
                fire_scatter(nfull, 0, tail)
                drain_scatter(0, tail)
            else:
                drain_scatter(0, RCH)
                drain_scatter(1, RCH)

        run_half(out_si, 0)
        run_half(out_sj, 1)

    return run(plut, pidx)


def kernel(grid_si, grid_sj, states_table, pos_table, ln_gamma, ln_beta):
    batch = grid_si.shape[0]
    ntok = grid_si.shape[1] * grid_si.shape[2]
    grid_cat = jnp.concatenate(
        [grid_si.reshape(batch, ntok), grid_sj.reshape(batch, ntok)], axis=0)
    plut4, idx2 = _prep(grid_cat, states_table, pos_table,
                        ln_gamma.reshape(1, HIDDEN), ln_beta.reshape(1, HIDDEN))
    plut = plut4.reshape(SEQ * NSTATES * NSTATES, 2 * HIDDEN)
    f = idx2.reshape(2, batch * SEQ // 2, 2)
    fa = f[:, :, 0]
    fb = f[:, :, 1]
    pidx = ((fa // NSTATES) * (NSTATES * NSTATES)
            + (fa % NSTATES) * NSTATES + (fb % NSTATES)).reshape(-1)
    out_si, out_sj = _sc_gather(plut, pidx, batch)
    return (out_si.reshape(batch, SEQ, HIDDEN),
            out_sj.reshape(batch, SEQ, HIDDEN))


# speedup vs baseline: 1.4608x; 1.4608x over previous
import functools

import jax
import jax.numpy as jnp
from jax import lax
from jax.experimental import pallas as pl
from jax.experimental.pallas import tpu as pltpu
from jax.experimental.pallas import tpu_sc as plsc

HIDDEN = 64
NSTATES = 8
SEQ = 197
EPS = 1e-5

NC, NS = 2, 16
NW = NC * NS
RCH = 480
GSL = [(o, min(128, RCH - o)) for o in range(0, RCH, 128)]


def _prep_body(g_ref, st_ref, pos_ref, gam_ref, bet_ref, plut_ref, idx_ref):
    pid = pl.program_id(0)
    g = g_ref[...]
    nb = g.shape[0]
    z1 = jnp.zeros((nb, 1), jnp.int32)
    m2 = jnp.concatenate(
        [z1, g[:, 0:SEQ - 1], z1, g[:, SEQ - 1:2 * (SEQ - 1)]], axis=1)
    ii = lax.broadcasted_iota(jnp.int32, (2 * SEQ, SEQ), 0)
    jj = lax.broadcasted_iota(jnp.int32, (2 * SEQ, SEQ), 1)
    w = (jnp.where(ii == 2 * jj, 8.0, 0.0)
         + jnp.where(ii == 2 * jj + 1, 1.0, 0.0)).astype(jnp.float32)
    mm = jnp.dot(m2.astype(jnp.float32), w,
                 preferred_element_type=jnp.float32)
    jq = lax.broadcasted_iota(jnp.int32, (nb, SEQ), 1)
    q = jnp.where(2 * jq < SEQ, 2 * jq, 2 * jq - SEQ)
    idx_ref[...] = q * (NSTATES * NSTATES) + mm.astype(jnp.int32)

    @pl.when(pid == 0)
    def _():
        x = (pos_ref[0:SEQ, :].reshape(SEQ, 1, HIDDEN)
             + st_ref[...].reshape(1, NSTATES, HIDDEN))
        mu = jnp.mean(x, axis=-1, keepdims=True)
        var = jnp.mean((x - mu) ** 2, axis=-1, keepdims=True)
        y = (x - mu) / jnp.sqrt(var + EPS)
        lut = (y * gam_ref[...].reshape(1, 1, HIDDEN)
               + bet_ref[...].reshape(1, 1, HIDDEN))
        lut_next = jnp.concatenate([lut[1:SEQ], lut[0:1]], axis=0)
        plut_ref[:, :, :, 0:HIDDEN] = jnp.broadcast_to(
            lut.reshape(SEQ, NSTATES, 1, HIDDEN),
            (SEQ, NSTATES, NSTATES, HIDDEN))
        plut_ref[:, :, :, HIDDEN:] = jnp.broadcast_to(
            lut_next.reshape(SEQ, 1, NSTATES, HIDDEN),
            (SEQ, NSTATES, NSTATES, HIDDEN))


def _prep(grid_cat, states_table, pos_table, gamma2, beta2):
    nrows = grid_cat.shape[0]
    nblk = 4
    blk = nrows // nblk
    return pl.pallas_call(
        _prep_body,
        grid=(nblk,),
        in_specs=[
            pl.BlockSpec((blk, grid_cat.shape[1]), lambda i: (i, 0)),
            pl.BlockSpec(states_table.shape, lambda i: (0, 0)),
            pl.BlockSpec(pos_table.shape, lambda i: (0, 0)),
            pl.BlockSpec((1, HIDDEN), lambda i: (0, 0)),
            pl.BlockSpec((1, HIDDEN), lambda i: (0, 0)),
        ],
        out_specs=[
            pl.BlockSpec((SEQ, NSTATES, NSTATES, 2 * HIDDEN),
                         lambda i: (0, 0, 0, 0)),
            pl.BlockSpec((blk, SEQ), lambda i: (i, 0)),
        ],
        out_shape=[
            jax.ShapeDtypeStruct((SEQ, NSTATES, NSTATES, 2 * HIDDEN),
                                 jnp.float32),
            jax.ShapeDtypeStruct((nrows, SEQ), jnp.int32),
        ],
    )(grid_cat, states_table, pos_table, gamma2, beta2)


def _sc_gather(plut, pidx, batch):
    pairs_per_tile = (batch * SEQ // 2) // NW
    nfull = (pairs_per_tile // RCH) & ~1
    tail = pairs_per_tile - nfull * RCH
    tail_gsl = [(o, min(128, tail - o)) for o in range(0, tail, 128)]
    mesh = plsc.VectorSubcoreMesh(core_axis_name="c", subcore_axis_name="s")
    out_t = jax.ShapeDtypeStruct((batch * SEQ // 2, 2 * HIDDEN), jnp.float32)

    @functools.partial(
        pl.kernel,
        out_type=(out_t, out_t),
        mesh=mesh,
        compiler_params=pltpu.CompilerParams(use_tc_tiling_on_sc=False),
        scratch_types=[
            pltpu.VMEM((RCH,), jnp.int32),
            pltpu.VMEM((RCH,), jnp.int32),
            pltpu.VMEM((RCH, 2 * HIDDEN), jnp.float32),
            pltpu.VMEM((RCH, 2 * HIDDEN), jnp.float32),
            pltpu.SemaphoreType.DMA,
            pltpu.SemaphoreType.DMA,
            pltpu.SemaphoreType.DMA,
            pltpu.SemaphoreType.DMA,
            pltpu.SemaphoreType.DMA,
            pltpu.SemaphoreType.DMA,
        ],
    )
    def run(plut_hbm, pidx_hbm, out_si, out_sj,
            iv0, iv1, rv0, rv1, si0, si1, sg0, sg1, ss0, ss1):
        wid = lax.axis_index("s") * NC + lax.axis_index("c")
        idx_v = (iv0, iv1)
        rows_v = (rv0, rv1)
        semi = (si0, si1)
        semg = (sg0, sg1)
        sems = (ss0, ss1)

        def run_half(out_ref, half):
            hrow0 = wid * pairs_per_tile
            irow0 = half * batch * SEQ // 2 + wid * pairs_per_tile

            def fire_idx(ci, b, n):
                pltpu.async_copy(pidx_hbm.at[pl.ds(irow0 + ci * RCH, n)],
                                 idx_v[b].at[pl.ds(0, n)], semi[b])

            def wait_idx(b, n):
                pltpu.make_async_copy(pidx_hbm.at[pl.ds(0, n)],
                                      idx_v[b].at[pl.ds(0, n)], semi[b]).wait()

            def fire_gathers(b, gsl):
                for (o, sz) in gsl:
                    pltpu.async_copy(plut_hbm.at[idx_v[b].at[pl.ds(o, sz)]],
                                     rows_v[b].at[pl.ds(o, sz)], semg[b])

            def wait_gathers(b, n):
                pltpu.make_async_copy(
                    plut_hbm.at[pl.ds(0, n)],
                    rows_v[b].at[pl.ds(0, n)], semg[b]).wait()

            def fire_scatter(ci, b, n):
                pltpu.async_copy(rows_v[b].at[pl.ds(0, n)],
                                 out_ref.at[pl.ds(hrow0 + ci * RCH, n)],
                                 sems[b])

            def drain_scatter(b, n):
                pltpu.make_async_copy(rows_v[b].at[pl.ds(0, n)],
                                      out_ref.at[pl.ds(0, n)], sems[b]).wait()

            fire_idx(0, 0, RCH)
            fire_idx(1, 1, RCH)

            @pl.loop(0, nfull, step=2)
            def _(g):
                for b in range(2):
                    gi = g + b
                    wait_idx(b, RCH)

                    @pl.when(gi >= 2)
                    def _():
                        drain_scatter(b, RCH)

                    fire_gathers(b, GSL)
                    wait_gathers(b, RCH)

                    @pl.when(gi + 2 < nfull)
                    def _():
                        fire_idx(gi + 2, b, RCH)

                    fire_scatter(gi, b, RCH)

            if tail:
                fire_idx(nfull, 0, tail)
                wait_idx(0, tail)
                drain_scatter(0, RCH)
                fire_gathers(0, tail_gsl)
                wait_gathers(0, tail)
                drain_scatter(1, RCH)
                fire_scatter(nfull, 0, tail)
                drain_scatter(0, tail)
            else:
                drain_scatter(0, RCH)
                drain_scatter(1, RCH)

        run_half(out_si, 0)
        run_half(out_sj, 1)

    return run(plut, pidx)


def kernel(grid_si, grid_sj, states_table, pos_table, ln_gamma, ln_beta):
    batch = grid_si.shape[0]
    ntok = grid_si.shape[1] * grid_si.shape[2]
    grid_cat = jnp.concatenate(
        [grid_si.reshape(batch, ntok), grid_sj.reshape(batch, ntok)], axis=0)
    grid2 = grid_cat.reshape(batch, 2 * ntok)
    plut4, pidx2 = _prep(grid2, states_table, pos_table,
                         ln_gamma.reshape(1, HIDDEN), ln_beta.reshape(1, HIDDEN))
    plut = plut4.reshape(SEQ * NSTATES * NSTATES, 2 * HIDDEN)
    pidx = pidx2.reshape(-1)
    out_si, out_sj = _sc_gather(plut, pidx, batch)
    return (out_si.reshape(batch, SEQ, HIDDEN),
            out_sj.reshape(batch, SEQ, HIDDEN))


# speedup vs baseline: 1.5153x; 1.0373x over previous
import functools

import jax
import jax.numpy as jnp
from jax import lax
from jax.experimental import pallas as pl
from jax.experimental.pallas import tpu as pltpu
from jax.experimental.pallas import tpu_sc as plsc

HIDDEN = 64
NSTATES = 8
SEQ = 197
EPS = 1e-5

NC, NS = 2, 16
NW = NC * NS
RCH = 480
GSL = [(o, min(128, RCH - o)) for o in range(0, RCH, 128)]


def _prep_body(g_ref, st_ref, pos_ref, gam_ref, bet_ref, plut_ref, idx_ref):
    pid = pl.program_id(0)
    g = g_ref[...]
    nb = g.shape[0]
    z1 = jnp.zeros((nb, 1), jnp.int32)
    m2 = jnp.concatenate(
        [z1, g[:, 0:SEQ - 1], z1, g[:, SEQ - 1:2 * (SEQ - 1)]], axis=1)
    ii = lax.broadcasted_iota(jnp.int32, (2 * SEQ, SEQ), 0)
    jj = lax.broadcasted_iota(jnp.int32, (2 * SEQ, SEQ), 1)
    w = (jnp.where(ii == 2 * jj, 8.0, 0.0)
         + jnp.where(ii == 2 * jj + 1, 1.0, 0.0)).astype(jnp.float32)
    mm = jnp.dot(m2.astype(jnp.float32), w,
                 preferred_element_type=jnp.float32)
    jq = lax.broadcasted_iota(jnp.int32, (nb, SEQ), 1)
    q = jnp.where(2 * jq < SEQ, 2 * jq, 2 * jq - SEQ)
    idx_ref[...] = q * (NSTATES * NSTATES) + mm.astype(jnp.int32)

    @pl.when(pid == 0)
    def _():
        x = (pos_ref[0:SEQ, :].reshape(SEQ, 1, HIDDEN)
             + st_ref[...].reshape(1, NSTATES, HIDDEN))
        mu = jnp.mean(x, axis=-1, keepdims=True)
        var = jnp.mean((x - mu) ** 2, axis=-1, keepdims=True)
        y = (x - mu) / jnp.sqrt(var + EPS)
        lut = (y * gam_ref[...].reshape(1, 1, HIDDEN)
               + bet_ref[...].reshape(1, 1, HIDDEN))
        lut_next = jnp.concatenate([lut[1:SEQ], lut[0:1]], axis=0)
        plut_ref[:, :, :, 0:HIDDEN] = jnp.broadcast_to(
            lut.reshape(SEQ, NSTATES, 1, HIDDEN),
            (SEQ, NSTATES, NSTATES, HIDDEN))
        plut_ref[:, :, :, HIDDEN:] = jnp.broadcast_to(
            lut_next.reshape(SEQ, 1, NSTATES, HIDDEN),
            (SEQ, NSTATES, NSTATES, HIDDEN))


def _prep(grid_cat, states_table, pos_table, gamma2, beta2):
    nrows = grid_cat.shape[0]
    nblk = 4
    blk = nrows // nblk
    return pl.pallas_call(
        _prep_body,
        grid=(nblk,),
        in_specs=[
            pl.BlockSpec((blk, grid_cat.shape[1]), lambda i: (i, 0)),
            pl.BlockSpec(states_table.shape, lambda i: (0, 0)),
            pl.BlockSpec(pos_table.shape, lambda i: (0, 0)),
            pl.BlockSpec((1, HIDDEN), lambda i: (0, 0)),
            pl.BlockSpec((1, HIDDEN), lambda i: (0, 0)),
        ],
        out_specs=[
            pl.BlockSpec((SEQ, NSTATES, NSTATES, 2 * HIDDEN),
                         lambda i: (0, 0, 0, 0)),
            pl.BlockSpec((blk, SEQ), lambda i: (i, 0)),
        ],
        out_shape=[
            jax.ShapeDtypeStruct((SEQ, NSTATES, NSTATES, 2 * HIDDEN),
                                 jnp.float32),
            jax.ShapeDtypeStruct((nrows, SEQ), jnp.int32),
        ],
    )(grid_cat, states_table, pos_table, gamma2, beta2)


def _sc_gather_one(plut, pidx, batch, half):
    pairs_per_tile = (batch * SEQ // 2) // NW
    nfull = (pairs_per_tile // RCH) & ~1
    tail = pairs_per_tile - nfull * RCH
    tail_gsl = [(o, min(128, tail - o)) for o in range(0, tail, 128)]
    mesh = plsc.VectorSubcoreMesh(core_axis_name="c", subcore_axis_name="s")
    out_t = jax.ShapeDtypeStruct((batch * SEQ // 2, 2 * HIDDEN), jnp.float32)

    @functools.partial(
        pl.kernel,
        out_type=out_t,
        mesh=mesh,
        compiler_params=pltpu.CompilerParams(use_tc_tiling_on_sc=False),
        scratch_types=[
            pltpu.VMEM((RCH,), jnp.int32),
            pltpu.VMEM((RCH,), jnp.int32),
            pltpu.VMEM((RCH, 2 * HIDDEN), jnp.float32),
            pltpu.VMEM((RCH, 2 * HIDDEN), jnp.float32),
            pltpu.SemaphoreType.DMA,
            pltpu.SemaphoreType.DMA,
            pltpu.SemaphoreType.DMA,
            pltpu.SemaphoreType.DMA,
            pltpu.SemaphoreType.DMA,
            pltpu.SemaphoreType.DMA,
        ],
    )
    def run(plut_hbm, pidx_hbm, out_ref,
            iv0, iv1, rv0, rv1, si0, si1, sg0, sg1, ss0, ss1):
        wid = lax.axis_index("s") * NC + lax.axis_index("c")
        idx_v = (iv0, iv1)
        rows_v = (rv0, rv1)
        semi = (si0, si1)
        semg = (sg0, sg1)
        sems = (ss0, ss1)

        hrow0 = wid * pairs_per_tile
        irow0 = half * batch * SEQ // 2 + wid * pairs_per_tile

        def fire_idx(ci, b, n):
            pltpu.async_copy(pidx_hbm.at[pl.ds(irow0 + ci * RCH, n)],
                             idx_v[b].at[pl.ds(0, n)], semi[b])

        def wait_idx(b, n):
            pltpu.make_async_copy(pidx_hbm.at[pl.ds(0, n)],
                                  idx_v[b].at[pl.ds(0, n)], semi[b]).wait()

        def fire_gathers(b, gsl):
            for (o, sz) in gsl:
                pltpu.async_copy(plut_hbm.at[idx_v[b].at[pl.ds(o, sz)]],
                                 rows_v[b].at[pl.ds(o, sz)], semg[b])

        def wait_gathers(b, n):
            pltpu.make_async_copy(
                plut_hbm.at[pl.ds(0, n)],
                rows_v[b].at[pl.ds(0, n)], semg[b]).wait()

        def fire_scatter(ci, b, n):
            pltpu.async_copy(rows_v[b].at[pl.ds(0, n)],
                             out_ref.at[pl.ds(hrow0 + ci * RCH, n)],
                             sems[b])

        def drain_scatter(b, n):
            pltpu.make_async_copy(rows_v[b].at[pl.ds(0, n)],
                                  out_ref.at[pl.ds(0, n)], sems[b]).wait()

        fire_idx(0, 0, RCH)
        fire_idx(1, 1, RCH)

        @pl.loop(0, nfull, step=2)
        def _(g):
            for b in range(2):
                gi = g + b
                wait_idx(b, RCH)

                @pl.when(gi >= 2)
                def _():
                    drain_scatter(b, RCH)

                fire_gathers(b, GSL)
                wait_gathers(b, RCH)

                @pl.when(gi + 2 < nfull)
                def _():
                    fire_idx(gi + 2, b, RCH)

                fire_scatter(gi, b, RCH)

        if tail:
            fire_idx(nfull, 0, tail)
            wait_idx(0, tail)
            drain_scatter(0, RCH)
            fire_gathers(0, tail_gsl)
            wait_gathers(0, tail)
            drain_scatter(1, RCH)
            fire_scatter(nfull, 0, tail)
            drain_scatter(0, tail)
        else:
            drain_scatter(0, RCH)
            drain_scatter(1, RCH)

    return run(plut, pidx)


def kernel(grid_si, grid_sj, states_table, pos_table, ln_gamma, ln_beta):
    batch = grid_si.shape[0]
    ntok = grid_si.shape[1] * grid_si.shape[2]
    grid_cat = jnp.concatenate(
        [grid_si.reshape(batch, ntok), grid_sj.reshape(batch, ntok)], axis=0)
    grid2 = grid_cat.reshape(batch, 2 * ntok)
    plut4, pidx2 = _prep(grid2, states_table, pos_table,
                         ln_gamma.reshape(1, HIDDEN), ln_beta.reshape(1, HIDDEN))
    plut = plut4.reshape(SEQ * NSTATES * NSTATES, 2 * HIDDEN)
    pidx = pidx2.reshape(-1)
    out_si = _sc_gather_one(plut, pidx, batch, 0)
    out_sj = _sc_gather_one(plut, pidx, batch, 1)
    return (out_si.reshape(batch, SEQ, HIDDEN),
            out_sj.reshape(batch, SEQ, HIDDEN))
